# baseline (speedup 1.0000x reference)
import functools

import jax
import jax.numpy as jnp
from jax import lax
from jax.experimental import pallas as pl
from jax.experimental.pallas import tpu as pltpu
from jax.experimental.pallas import tpu_sc as plsc

_NUM_ITEMS = 1000000
_BATCH = 16384
_HIST = 200
_N = _BATCH * _HIST

_TROWS = 8
_TCOLS = 131072
_PAD_N = _TROWS * _TCOLS

_NC, _NS = 2, 16
_NW = _NC * _NS
_ROWS_PER_W = _BATCH // _NW
_CROWS = 64
_CHUNK = _CROWS * _HIST
_NCHUNKS = _ROWS_PER_W // _CROWS

_COLS = tuple(range(0, _HIST - 8, 16))
_TAIL = _HIST - 16

_TSLICE = _PAD_N // _NS


def _sigmoid_body(u_ref, x_ref, o_ref):
    o_ref[...] = jax.nn.sigmoid(u_ref[0] - x_ref[...])


def _sigmoid_table(user_params, table_pad):
    return pl.pallas_call(
        _sigmoid_body,
        grid=(16,),
        out_shape=jax.ShapeDtypeStruct((_TROWS, _TCOLS), jnp.float32),
        in_specs=[
            pl.BlockSpec(memory_space=pltpu.SMEM),
            pl.BlockSpec((_TROWS, _TCOLS // 16), lambda i: (0, i)),
        ],
        out_specs=pl.BlockSpec((_TROWS, _TCOLS // 16), lambda i: (0, i)),
    )(user_params, table_pad)


def _gather_body(tbl_hbm, idx_hbm, out_hbm, tbl_sh, iv2, iv1, rv1, rv2, sem):
    wid = lax.axis_index("s") * _NC + lax.axis_index("c")
    sid = lax.axis_index("s")
    trow = sid % _TROWS
    thalf = sid // _TROWS
    tcols = _TCOLS // 2
    pltpu.sync_copy(
        tbl_hbm.at[trow, pl.ds(thalf * tcols, tcols)],
        tbl_sh.at[pl.ds((trow * 2 + thalf) * tcols, tcols)],
    )
    plsc.subcore_barrier()

    def pack_rows(r, _):
        for c in _COLS:
            iv1[pl.ds(r * _HIST + c, 16)] = iv2[r, pl.ds(c, 16)]
        c_dyn = r * 0 + _TAIL
        iv1[pl.ds(r * _HIST + _TAIL, 16)] = iv2[r, pl.ds(c_dyn, 16)]
        return _

    def unpack_rows(r, _):
        c_dyn = r * 0 + _TAIL
        rv2[r, pl.ds(c_dyn, 16)] = rv1[pl.ds(r * _HIST + _TAIL, 16)]
        for c in _COLS:
            rv2[r, pl.ds(c, 16)] = rv1[pl.ds(r * _HIST + c, 16)]
        return _

    for k in range(_NCHUNKS):
        r0 = wid * _ROWS_PER_W + k * _CROWS
        pltpu.sync_copy(idx_hbm.at[pl.ds(r0, _CROWS), :], iv2)
        lax.fori_loop(0, _CROWS, pack_rows, 0)
        pltpu.async_copy(tbl_sh.at[iv1], rv1, sem).wait()
        lax.fori_loop(0, _CROWS, unpack_rows, 0)
        pltpu.sync_copy(rv2, out_hbm.at[pl.ds(r0, _CROWS), :])


@functools.cache
def _make_gather():
    return pl.kernel(
        _gather_body,
        mesh=plsc.VectorSubcoreMesh(core_axis_name="c", subcore_axis_name="s"),
        out_type=jax.ShapeDtypeStruct((_BATCH, _HIST), jnp.float32),
        scratch_types=[
            pltpu.VMEM_SHARED((_PAD_N,), jnp.float32),
            pltpu.VMEM((_CROWS, _HIST), jnp.int32),
            pltpu.VMEM((_CHUNK,), jnp.int32),
            pltpu.VMEM((_CHUNK,), jnp.float32),
            pltpu.VMEM((_CROWS, _HIST), jnp.float32),
            pltpu.SemaphoreType.DMA,
        ],
    )


def kernel(item_nos, user_params, item_table):
    idx = item_nos.astype(jnp.int32)
    t = item_table.reshape(-1)
    t_pad = jnp.pad(t, (0, _PAD_N - _NUM_ITEMS)).reshape(_TROWS, _TCOLS)
    f = _sigmoid_table(user_params, t_pad)
    return _make_gather()(f, idx)
